# baseline (speedup 1.0000x reference)
import jax
import jax.numpy as jnp
from jax import lax
from jax.experimental import pallas as pl
from jax.experimental.pallas import tpu as pltpu
from jax.experimental.pallas import tpu_sc as plsc

_B = 128
_LANES = 16
_NC = 2
_NS = 16


def _build_sc_call(n_atoms, interpret=False):
    NW = _NC * _NS
    PER = -(-n_atoms // NW)
    PER = -(-PER // _LANES) * _LANES
    while PER % 8:
        PER += _LANES
    LAST_BASE = (NW - 1) * PER
    LAST = n_atoms - LAST_BASE
    assert LAST > 0 and LAST % _LANES == 0
    assert n_atoms % 8 == 0

    mesh = plsc.VectorSubcoreMesh(
        core_axis_name="c", subcore_axis_name="s",
        num_cores=_NC, num_subcores=_NS)

    def body(pos_hbm, bid_hbm, out_hbm, pos_v, bid_v, acc_v, idx_v, shared,
             sem1, sem2):
        cid = lax.axis_index("c")
        sid = lax.axis_index("s")
        wid = cid * _NS + sid
        lane = lax.iota(jnp.int32, _LANES)
        is_last = wid == (NW - 1)
        base = wid * PER

        def copies(sz):
            cps = [
                pltpu.make_async_copy(
                    pos_hbm.at[pl.ds(c * n_atoms + base, sz)],
                    pos_v.at[pl.ds(c * PER, sz)], sem1)
                for c in range(3)
            ]
            cps.append(pltpu.make_async_copy(
                bid_hbm.at[pl.ds(base, sz)], bid_v.at[pl.ds(0, sz)], sem2))
            return cps

        @pl.when(~is_last)
        def _stage_full():
            for cp in copies(PER):
                cp.start()

        @pl.when(is_last)
        def _stage_tail():
            for cp in copies(LAST):
                cp.start()

        for k in range(_B // _LANES):
            acc_v[pl.ds(k * _LANES, _LANES)] = jnp.zeros((_LANES,), jnp.float32)
            idx_v[pl.ds(k * _LANES, _LANES)] = lane + (k * _LANES)

        @pl.when(sid == 0)
        def _zero_shared():
            pltpu.sync_copy(acc_v, shared)

        plsc.subcore_barrier()

        @pl.when(~is_last)
        def _wait_full():
            for cp in copies(PER):
                cp.wait()

        @pl.when(is_last)
        def _wait_tail():
            for cp in copies(LAST):
                cp.wait()

        nblocks = jnp.where(is_last, LAST // _LANES, PER // _LANES)
        last_lane = lane == (_LANES - 1)

        @plsc.parallel_loop(0, nblocks, 1, unroll=8)
        def _block(j):
            a0 = j * _LANES
            bid = bid_v[pl.ds(a0, _LANES)]
            bidn = bid_v[pl.ds(a0 + 1, _LANES)]
            x = pos_v[pl.ds(a0, _LANES)]
            y = pos_v[pl.ds(PER + a0, _LANES)]
            z = pos_v[pl.ds(2 * PER + a0, _LANES)]
            s = plsc.cumsum(x * x + y * y + z * z)
            neq = bid != bidn
            plsc.addupdate_scatter(acc_v, [bid], s, mask=neq | last_lane)
            plsc.addupdate_scatter(acc_v, [bidn], -s,
                                   mask=neq & (~last_lane))

        pltpu.sync_copy(acc_v, shared.at[idx_v], add=True)
        plsc.subcore_barrier()

        @pl.when(sid == 0)
        def _write_out():
            pltpu.sync_copy(shared, out_hbm.at[cid])

    return pl.kernel(
        body,
        out_type=jax.ShapeDtypeStruct((_NC, _B), jnp.float32),
        mesh=mesh,
        scratch_types=[
            pltpu.VMEM((3 * PER,), jnp.float32),
            pltpu.VMEM((PER + _LANES,), jnp.int32),
            pltpu.VMEM((_B,), jnp.float32),
            pltpu.VMEM((_B,), jnp.int32),
            pltpu.VMEM_SHARED((_B,), jnp.float32),
            pltpu.SemaphoreType.DMA,
            pltpu.SemaphoreType.DMA,
        ],
        compiler_params=pltpu.CompilerParams(
            needs_layout_passes=False,
            disable_bounds_checks=True,
            disable_semaphore_checks=True,
        ),
        interpret=interpret,
    )


def kernel(positions, batch_idx, num_graphs):
    del num_graphs
    call = _build_sc_call(positions.shape[0])
    part = call(positions.T.reshape(-1), batch_idx.astype(jnp.int32))
    return (part[0] + part[1]).reshape(_B, 1)


# speedup vs baseline: 1.0636x; 1.0636x over previous
import jax
import jax.numpy as jnp
from jax import lax
from jax.experimental import pallas as pl
from jax.experimental.pallas import tpu as pltpu
from jax.experimental.pallas import tpu_sc as plsc

_B = 128
_LANES = 16


def _build_sc_call(n_atoms, interpret=False):
    NW = 16
    PER = -(-n_atoms // NW)
    PER = -(-PER // _LANES) * _LANES
    while PER % 8:
        PER += _LANES
    LAST_BASE = (NW - 1) * PER
    LAST = n_atoms - LAST_BASE
    assert LAST > 0 and LAST % _LANES == 0
    assert n_atoms % 8 == 0

    mesh = plsc.VectorSubcoreMesh(
        core_axis_name="c", subcore_axis_name="s",
        num_cores=1, num_subcores=NW)

    def body(pos_hbm, bid_hbm, out_hbm, pos_v, bid_v, acc_v, idx_v, shared,
             sem1, sem2):
        wid = lax.axis_index("s")
        lane = lax.iota(jnp.int32, _LANES)
        is_last = wid == (NW - 1)
        base = wid * PER

        def copies(sz):
            cps = [
                pltpu.make_async_copy(
                    pos_hbm.at[pl.ds(c * n_atoms + base, sz)],
                    pos_v.at[pl.ds(c * PER, sz)], sem1)
                for c in range(3)
            ]
            cps.append(pltpu.make_async_copy(
                bid_hbm.at[pl.ds(base, sz)], bid_v.at[pl.ds(0, sz)], sem2))
            return cps

        @pl.when(~is_last)
        def _stage_full():
            for cp in copies(PER):
                cp.start()

        @pl.when(is_last)
        def _stage_tail():
            for cp in copies(LAST):
                cp.start()

        for k in range(_B // _LANES):
            acc_v[pl.ds(k * _LANES, _LANES)] = jnp.zeros((_LANES,), jnp.float32)
            idx_v[pl.ds(k * _LANES, _LANES)] = lane + (k * _LANES)

        @pl.when(wid == 0)
        def _zero_shared():
            pltpu.sync_copy(acc_v, shared)

        plsc.subcore_barrier()

        @pl.when(~is_last)
        def _wait_full():
            for cp in copies(PER):
                cp.wait()

        @pl.when(is_last)
        def _wait_tail():
            for cp in copies(LAST):
                cp.wait()

        nblocks = jnp.where(is_last, LAST // _LANES, PER // _LANES)
        last_lane = lane == (_LANES - 1)

        @plsc.parallel_loop(0, nblocks, 1, unroll=8)
        def _block(j):
            a0 = j * _LANES
            bid = bid_v[pl.ds(a0, _LANES)]
            bidn = bid_v[pl.ds(a0 + 1, _LANES)]
            x = pos_v[pl.ds(a0, _LANES)]
            y = pos_v[pl.ds(PER + a0, _LANES)]
            z = pos_v[pl.ds(2 * PER + a0, _LANES)]
            s = plsc.cumsum(x * x + y * y + z * z)
            neq = bid != bidn
            plsc.addupdate_scatter(acc_v, [bid], s, mask=neq | last_lane)
            plsc.addupdate_scatter(acc_v, [bidn], -s,
                                   mask=neq & (~last_lane))

        pltpu.sync_copy(acc_v, shared.at[idx_v], add=True)
        plsc.subcore_barrier()

        @pl.when(wid == 0)
        def _write_out():
            pltpu.sync_copy(shared, out_hbm)

    return pl.kernel(
        body,
        out_type=jax.ShapeDtypeStruct((_B,), jnp.float32),
        mesh=mesh,
        scratch_types=[
            pltpu.VMEM((3 * PER,), jnp.float32),
            pltpu.VMEM((PER + _LANES,), jnp.int32),
            pltpu.VMEM((_B,), jnp.float32),
            pltpu.VMEM((_B,), jnp.int32),
            pltpu.VMEM_SHARED((_B,), jnp.float32),
            pltpu.SemaphoreType.DMA,
            pltpu.SemaphoreType.DMA,
        ],
        compiler_params=pltpu.CompilerParams(
            needs_layout_passes=False,
            disable_bounds_checks=True,
            disable_semaphore_checks=True,
            skip_device_barrier=True,
        ),
        interpret=interpret,
    )


def kernel(positions, batch_idx, num_graphs):
    del num_graphs
    call = _build_sc_call(positions.shape[0])
    out = call(positions.T.reshape(-1), batch_idx.astype(jnp.int32))
    return out.reshape(_B, 1)


# speedup vs baseline: 1.0661x; 1.0024x over previous
import jax
import jax.numpy as jnp
from jax import lax
from jax.experimental import pallas as pl
from jax.experimental.pallas import tpu as pltpu
from jax.experimental.pallas import tpu_sc as plsc

_B = 128
_LANES = 16


def _build_sc_call(n_atoms, interpret=False):
    NW = 16
    PER = -(-n_atoms // NW)
    PER = -(-PER // _LANES) * _LANES
    while PER % 8:
        PER += _LANES
    LAST_BASE = (NW - 1) * PER
    LAST = n_atoms - LAST_BASE
    assert LAST > 0 and LAST % _LANES == 0
    assert n_atoms % 8 == 0

    mesh = plsc.VectorSubcoreMesh(
        core_axis_name="c", subcore_axis_name="s",
        num_cores=1, num_subcores=NW)

    def body(pos_hbm, bid_hbm, out_hbm, pos_v, bid_v, acc_v, idx_v, shared,
             sem1, sem2):
        wid = lax.axis_index("s")
        lane = lax.iota(jnp.int32, _LANES)
        is_last = wid == (NW - 1)
        base = wid * PER

        def copies(sz):
            cps = [
                pltpu.make_async_copy(
                    pos_hbm.at[pl.ds(c * n_atoms + base, sz)],
                    pos_v.at[pl.ds(c * PER, sz)], sem1)
                for c in range(3)
            ]
            cps.append(pltpu.make_async_copy(
                bid_hbm.at[pl.ds(base, sz)], bid_v.at[pl.ds(0, sz)], sem2))
            return cps

        @pl.when(~is_last)
        def _stage_full():
            for cp in copies(PER):
                cp.start()

        @pl.when(is_last)
        def _stage_tail():
            for cp in copies(LAST):
                cp.start()

        for k in range(_B // _LANES):
            acc_v[pl.ds(k * _LANES, _LANES)] = jnp.zeros((_LANES,), jnp.float32)
            idx_v[pl.ds(k * _LANES, _LANES)] = lane + (k * _LANES)

        @pl.when(wid == 0)
        def _zero_shared():
            pltpu.sync_copy(acc_v, shared)

        plsc.subcore_barrier()

        @pl.when(~is_last)
        def _wait_full():
            for cp in copies(PER):
                cp.wait()

        @pl.when(is_last)
        def _wait_tail():
            for cp in copies(LAST):
                cp.wait()

        nblocks = jnp.where(is_last, LAST // _LANES, PER // _LANES)
        last_lane = lane == (_LANES - 1)

        @plsc.parallel_loop(0, nblocks, 1, unroll=8)
        def _block(j):
            a0 = j * _LANES
            bid = bid_v[pl.ds(a0, _LANES)]
            bidn = bid_v[pl.ds(a0 + 1, _LANES)]
            x = pos_v[pl.ds(a0, _LANES)]
            y = pos_v[pl.ds(PER + a0, _LANES)]
            z = pos_v[pl.ds(2 * PER + a0, _LANES)]
            s = plsc.cumsum(x * x + y * y + z * z)
            neq = bid != bidn
            plsc.addupdate_scatter(acc_v, [bid], s, mask=neq | last_lane)
            plsc.addupdate_scatter(acc_v, [bidn], -s,
                                   mask=neq & (~last_lane))

        pltpu.sync_copy(acc_v, shared.at[idx_v], add=True)
        plsc.subcore_barrier()

        @pl.when(wid == 0)
        def _write_out():
            pltpu.sync_copy(shared, out_hbm)

    return pl.kernel(
        body,
        out_type=jax.ShapeDtypeStruct((_B,), jnp.float32),
        mesh=mesh,
        scratch_types=[
            pltpu.VMEM((3 * PER,), jnp.float32),
            pltpu.VMEM((PER + _LANES,), jnp.int32),
            pltpu.VMEM((_B,), jnp.float32),
            pltpu.VMEM((_B,), jnp.int32),
            pltpu.VMEM_SHARED((_B,), jnp.float32),
            pltpu.SemaphoreType.DMA,
            pltpu.SemaphoreType.DMA,
        ],
        compiler_params=pltpu.CompilerParams(needs_layout_passes=False),
        interpret=interpret,
    )


def kernel(positions, batch_idx, num_graphs):
    del num_graphs
    call = _build_sc_call(positions.shape[0])
    out = call(positions.T.reshape(-1), batch_idx.astype(jnp.int32))
    return out.reshape(_B, 1)
